# baseline (speedup 1.0000x reference)
import functools
import math

import jax
import jax.numpy as jnp
from jax import lax
from jax.experimental import pallas as pl
from jax.experimental.pallas import tpu as pltpu
from jax.experimental.pallas import tpu_sc as plsc

N = 10000
E = 160000
IN_C = 256
HID = 512
OUT_C = 64
NUM_LAYERS = 8
ALPHA = 0.5
THETA = 1.0
NUM_GRAPHS = 128

FC = 128
NCHUNK = HID // FC
NTILE = 16
BATCH = 256
EPT = 10240
NBATCH = EPT // BATCH
E_PAD = EPT * NTILE
N_PAD = 10008
AGG_ROWS = N_PAD
ZMAIN = 624
ZLAST = N_PAD - 15 * ZMAIN

ROW_BLK = 1112
GRID = N_PAD // ROW_BLK


def _spmm_body(*args):
    h_refs = args[0:NCHUNK]
    pkb, zeros = args[NCHUNK:NCHUNK + 2]
    o_refs = args[NCHUNK + 2:2 * NCHUNK + 2]
    pk_v, su, du, rows, gsem, ssem, agg = args[2 * NCHUNK + 2:]
    cid = lax.axis_index("c")
    sid = lax.axis_index("s")
    pltpu.sync_copy(pkb.at[sid], pk_v)

    for chunk in range(NCHUNK):
        @pl.when(cid == chunk // (NCHUNK // 2))
        def _(h_ref=h_refs[chunk], o_ref=o_refs[chunk]):
            @pl.when(sid < NTILE - 1)
            def _zero_main():
                pltpu.sync_copy(zeros.at[pl.ds(sid * ZMAIN, ZMAIN)],
                                agg.at[pl.ds(sid * ZMAIN, ZMAIN)])

            @pl.when(sid == NTILE - 1)
            def _zero_last():
                pltpu.sync_copy(zeros.at[pl.ds(15 * ZMAIN, ZLAST)],
                                agg.at[pl.ds(15 * ZMAIN, ZLAST)])

            plsc.subcore_barrier()

            def batch_body(j, carry):
                for i in range(BATCH // 16):
                    pk = pk_v[j, pl.ds(i * 16, 16)]
                    su[0, pl.ds(i * 16, 16)] = pk & 0xFFFF
                    du[0, pl.ds(i * 16, 16)] = pk >> 16
                pltpu.async_copy(h_ref.at[su.at[0]], rows, gsem).wait()
                pltpu.sync_copy(rows, agg.at[du.at[0]], add=True)
                return carry

            lax.fori_loop(0, NBATCH, batch_body, 0)

            plsc.subcore_barrier()

            @pl.when(sid < NTILE - 1)
            def _copy_main():
                pltpu.sync_copy(agg.at[pl.ds(sid * ZMAIN, ZMAIN)],
                                o_ref.at[pl.ds(sid * ZMAIN, ZMAIN)])

            @pl.when(sid == NTILE - 1)
            def _copy_last():
                pltpu.sync_copy(agg.at[pl.ds(15 * ZMAIN, ZLAST)],
                                o_ref.at[pl.ds(15 * ZMAIN, ZLAST)])

            plsc.subcore_barrier()


_spmm_call = pl.kernel(
    _spmm_body,
    out_type=tuple(jax.ShapeDtypeStruct((N_PAD, FC), jnp.float32)
                   for _ in range(NCHUNK)),
    mesh=plsc.VectorSubcoreMesh(core_axis_name="c", subcore_axis_name="s"),
    scratch_types=[
        pltpu.VMEM((NBATCH, BATCH), jnp.int32),
        pltpu.VMEM((1, BATCH), jnp.int32),
        pltpu.VMEM((1, BATCH), jnp.int32),
        pltpu.VMEM((BATCH, FC), jnp.float32),
        pltpu.SemaphoreType.DMA,
        pltpu.SemaphoreType.DMA,
        pltpu.VMEM_SHARED((AGG_ROWS, FC), jnp.float32),
    ],
)


def _split_store(o_refs, y):
    for j, o in enumerate(o_refs):
        o[...] = y[:, j * FC:(j + 1) * FC]


def _cat(refs):
    return jnp.concatenate([r[...] for r in refs], axis=1)


def _lin0_body(x_ref, w_ref, b_ref, *o_refs):
    y = jnp.maximum(
        jnp.dot(x_ref[...], w_ref[...], preferred_element_type=jnp.float32)
        + b_ref[...], 0.0)
    _split_store(o_refs, y)


def _lin0(x, w, b):
    return pl.pallas_call(
        _lin0_body,
        grid=(GRID,),
        in_specs=[
            pl.BlockSpec((ROW_BLK, IN_C), lambda i: (i, 0)),
            pl.BlockSpec((IN_C, HID), lambda i: (0, 0)),
            pl.BlockSpec((1, HID), lambda i: (0, 0)),
        ],
        out_specs=[pl.BlockSpec((ROW_BLK, FC), lambda i: (i, 0))] * NCHUNK,
        out_shape=[jax.ShapeDtypeStruct((N_PAD, FC), jnp.float32)] * NCHUNK,
    )(x, w, b.reshape(1, HID))


def _layer_body(beta, *refs):
    agg_refs = refs[0:NCHUNK]
    x0_refs = refs[NCHUNK:2 * NCHUNK]
    h_refs = refs[2 * NCHUNK:3 * NCHUNK]
    w_ref = refs[3 * NCHUNK]
    o_refs = refs[3 * NCHUNK + 1:]
    out = _cat(agg_refs) * (1.0 - ALPHA) + ALPHA * _cat(x0_refs)
    y = (1.0 - beta) * out + beta * jnp.dot(
        out, w_ref[...], preferred_element_type=jnp.float32)
    _split_store(o_refs, jnp.maximum(y + _cat(h_refs), 0.0))


def _layer(aggs, x0s, hs, w, beta):
    blk = pl.BlockSpec((ROW_BLK, FC), lambda i: (i, 0))
    return pl.pallas_call(
        functools.partial(_layer_body, beta),
        grid=(GRID,),
        in_specs=[blk] * (3 * NCHUNK)
        + [pl.BlockSpec((HID, HID), lambda i: (0, 0))],
        out_specs=[blk] * NCHUNK,
        out_shape=[jax.ShapeDtypeStruct((N_PAD, FC), jnp.float32)] * NCHUNK,
    )(*aggs, *x0s, *hs, w)


def _pool_head_body(*refs):
    h_refs = refs[0:NCHUNK]
    batch_ref, w_ref, b_ref, o_ref, sums, counts = refs[NCHUNK:]
    i = pl.program_id(0)

    @pl.when(i == 0)
    def _init():
        sums[...] = jnp.zeros_like(sums)
        counts[...] = jnp.zeros_like(counts)

    seg = batch_ref[0]
    gids = lax.broadcasted_iota(jnp.int32, (NUM_GRAPHS, ROW_BLK), 0)
    onehot = (gids == seg).astype(jnp.float32)
    sums[...] += jnp.dot(onehot, _cat(h_refs),
                         preferred_element_type=jnp.float32)
    counts[...] += jnp.sum(onehot, axis=1, keepdims=True)

    @pl.when(i == GRID - 1)
    def _fin():
        pooled = sums[...] / jnp.clip(counts[...], 1.0, None)
        logits = jnp.dot(pooled, w_ref[...],
                         preferred_element_type=jnp.float32) + b_ref[...]
        m = jnp.max(logits, axis=-1, keepdims=True)
        z = logits - m
        lse = jnp.log(jnp.sum(jnp.exp(z), axis=-1, keepdims=True))
        o_ref[...] = z - lse


def _pool_head(hs, batch, w, b):
    blk = pl.BlockSpec((ROW_BLK, FC), lambda i: (i, 0))
    return pl.pallas_call(
        _pool_head_body,
        grid=(GRID,),
        in_specs=[blk] * NCHUNK + [
            pl.BlockSpec((1, 1, ROW_BLK), lambda i: (i, 0, 0)),
            pl.BlockSpec((HID, OUT_C), lambda i: (0, 0)),
            pl.BlockSpec((1, OUT_C), lambda i: (0, 0)),
        ],
        out_specs=pl.BlockSpec((NUM_GRAPHS, OUT_C), lambda i: (0, 0)),
        out_shape=jax.ShapeDtypeStruct((NUM_GRAPHS, OUT_C), jnp.float32),
        scratch_shapes=[
            pltpu.VMEM((NUM_GRAPHS, HID), jnp.float32),
            pltpu.VMEM((NUM_GRAPHS, 1), jnp.float32),
        ],
    )(*hs, batch.reshape(GRID, 1, ROW_BLK), w, b.reshape(1, OUT_C))


def kernel(x, edge_index, batch, lin0_w, lin0_b, conv_ws, lin1_w, lin1_b):
    src = edge_index[0]
    dst = edge_index[1]
    npad = E_PAD - E
    srcp = jnp.concatenate([src, jnp.full((npad,), N, jnp.int32)])
    dstp = jnp.concatenate([dst, jnp.zeros((npad,), jnp.int32)])
    pkb = ((dstp << 16) | srcp).reshape(NTILE, NBATCH, BATCH)
    zeros = jnp.zeros((AGG_ROWS, FC), jnp.float32)

    x_pad = jnp.zeros((N_PAD, IN_C), jnp.float32).at[:N].set(x)
    batch_pad = jnp.concatenate(
        [batch, jnp.full((N_PAD - N,), -1, jnp.int32)])

    hs = _lin0(x_pad, lin0_w, lin0_b)
    x0s = hs
    for layer in range(NUM_LAYERS):
        beta = float(math.log(THETA / (layer + 1) + 1.0))
        aggs = _spmm_call(*hs, pkb, zeros)
        hs = _layer(aggs, x0s, hs, conv_ws[layer], beta)
    return _pool_head(hs, batch_pad, lin1_w, lin1_b)


# speedup vs baseline: 1.2842x; 1.2842x over previous
import functools
import math

import jax
import jax.numpy as jnp
from jax import lax
from jax.experimental import pallas as pl
from jax.experimental.pallas import tpu as pltpu
from jax.experimental.pallas import tpu_sc as plsc

N = 10000
E = 160000
IN_C = 256
HID = 512
OUT_C = 64
NUM_LAYERS = 8
ALPHA = 0.5
THETA = 1.0
NUM_GRAPHS = 128

FC = 128
NCHUNK = HID // FC
NTILE = 16
BATCH = 128
EPT = 10112
NBATCH = EPT // BATCH
E_PAD = EPT * NTILE
N_PAD = 10008
AGG_ROWS = N_PAD
ZMAIN = 624
ZLAST = N_PAD - 15 * ZMAIN

ROW_BLK = 1112
GRID = N_PAD // ROW_BLK


def _spmm_body(*args):
    h_refs = args[0:NCHUNK]
    srcb, dstb, zeros = args[NCHUNK:NCHUNK + 3]
    o_refs = args[NCHUNK + 3:2 * NCHUNK + 3]
    src_v, dst_v, rows, gsem, agg = args[2 * NCHUNK + 3:]
    cid = lax.axis_index("c")
    sid = lax.axis_index("s")
    pltpu.sync_copy(srcb.at[sid], src_v)
    pltpu.sync_copy(dstb.at[sid], dst_v)

    for chunk in range(NCHUNK):
        @pl.when(cid == chunk // (NCHUNK // 2))
        def _(h_ref=h_refs[chunk], o_ref=o_refs[chunk]):
            @pl.when(sid < NTILE - 1)
            def _zero_main():
                pltpu.sync_copy(zeros.at[pl.ds(sid * ZMAIN, ZMAIN)],
                                agg.at[pl.ds(sid * ZMAIN, ZMAIN)])

            @pl.when(sid == NTILE - 1)
            def _zero_last():
                pltpu.sync_copy(zeros.at[pl.ds(15 * ZMAIN, ZLAST)],
                                agg.at[pl.ds(15 * ZMAIN, ZLAST)])

            plsc.subcore_barrier()

            def batch_body(j, carry):
                pltpu.async_copy(h_ref.at[src_v.at[j]], rows, gsem).wait()
                pltpu.sync_copy(rows, agg.at[dst_v.at[j]], add=True)
                return carry

            lax.fori_loop(0, NBATCH, batch_body, 0)

            plsc.subcore_barrier()

            @pl.when(sid < NTILE - 1)
            def _copy_main():
                pltpu.sync_copy(agg.at[pl.ds(sid * ZMAIN, ZMAIN)],
                                o_ref.at[pl.ds(sid * ZMAIN, ZMAIN)])

            @pl.when(sid == NTILE - 1)
            def _copy_last():
                pltpu.sync_copy(agg.at[pl.ds(15 * ZMAIN, ZLAST)],
                                o_ref.at[pl.ds(15 * ZMAIN, ZLAST)])

            plsc.subcore_barrier()


_spmm_call = pl.kernel(
    _spmm_body,
    out_type=tuple(jax.ShapeDtypeStruct((N_PAD, FC), jnp.float32)
                   for _ in range(NCHUNK)),
    mesh=plsc.VectorSubcoreMesh(core_axis_name="c", subcore_axis_name="s"),
    scratch_types=[
        pltpu.VMEM((NBATCH, BATCH), jnp.int32),
        pltpu.VMEM((NBATCH, BATCH), jnp.int32),
        pltpu.VMEM((BATCH, FC), jnp.float32),
        pltpu.SemaphoreType.DMA,
        pltpu.VMEM_SHARED((AGG_ROWS, FC), jnp.float32),
    ],
)


def _split_store(o_refs, y):
    for j, o in enumerate(o_refs):
        o[...] = y[:, j * FC:(j + 1) * FC]


def _cat(refs):
    return jnp.concatenate([r[...] for r in refs], axis=1)


def _lin0_body(x_ref, w_ref, b_ref, *o_refs):
    y = jnp.maximum(
        jnp.dot(x_ref[...], w_ref[...], preferred_element_type=jnp.float32)
        + b_ref[...], 0.0)
    _split_store(o_refs, y)


def _lin0(x, w, b):
    return pl.pallas_call(
        _lin0_body,
        grid=(GRID,),
        in_specs=[
            pl.BlockSpec((ROW_BLK, IN_C), lambda i: (i, 0)),
            pl.BlockSpec((IN_C, HID), lambda i: (0, 0)),
            pl.BlockSpec((1, HID), lambda i: (0, 0)),
        ],
        out_specs=[pl.BlockSpec((ROW_BLK, FC), lambda i: (i, 0))] * NCHUNK,
        out_shape=[jax.ShapeDtypeStruct((N_PAD, FC), jnp.float32)] * NCHUNK,
    )(x, w, b.reshape(1, HID))


def _layer_body(beta, *refs):
    agg_refs = refs[0:NCHUNK]
    x0_refs = refs[NCHUNK:2 * NCHUNK]
    h_refs = refs[2 * NCHUNK:3 * NCHUNK]
    w_ref = refs[3 * NCHUNK]
    o_refs = refs[3 * NCHUNK + 1:]
    out = _cat(agg_refs) * (1.0 - ALPHA) + ALPHA * _cat(x0_refs)
    y = (1.0 - beta) * out + beta * jnp.dot(
        out, w_ref[...], preferred_element_type=jnp.float32)
    _split_store(o_refs, jnp.maximum(y + _cat(h_refs), 0.0))


def _layer(aggs, x0s, hs, w, beta):
    blk = pl.BlockSpec((ROW_BLK, FC), lambda i: (i, 0))
    return pl.pallas_call(
        functools.partial(_layer_body, beta),
        grid=(GRID,),
        in_specs=[blk] * (3 * NCHUNK)
        + [pl.BlockSpec((HID, HID), lambda i: (0, 0))],
        out_specs=[blk] * NCHUNK,
        out_shape=[jax.ShapeDtypeStruct((N_PAD, FC), jnp.float32)] * NCHUNK,
    )(*aggs, *x0s, *hs, w)


def _pool_head_body(*refs):
    h_refs = refs[0:NCHUNK]
    batch_ref, w_ref, b_ref, o_ref, sums, counts = refs[NCHUNK:]
    i = pl.program_id(0)

    @pl.when(i == 0)
    def _init():
        sums[...] = jnp.zeros_like(sums)
        counts[...] = jnp.zeros_like(counts)

    seg = batch_ref[0]
    gids = lax.broadcasted_iota(jnp.int32, (NUM_GRAPHS, ROW_BLK), 0)
    onehot = (gids == seg).astype(jnp.float32)
    sums[...] += jnp.dot(onehot, _cat(h_refs),
                         preferred_element_type=jnp.float32)
    counts[...] += jnp.sum(onehot, axis=1, keepdims=True)

    @pl.when(i == GRID - 1)
    def _fin():
        pooled = sums[...] / jnp.clip(counts[...], 1.0, None)
        logits = jnp.dot(pooled, w_ref[...],
                         preferred_element_type=jnp.float32) + b_ref[...]
        m = jnp.max(logits, axis=-1, keepdims=True)
        z = logits - m
        lse = jnp.log(jnp.sum(jnp.exp(z), axis=-1, keepdims=True))
        o_ref[...] = z - lse


def _pool_head(hs, batch, w, b):
    blk = pl.BlockSpec((ROW_BLK, FC), lambda i: (i, 0))
    return pl.pallas_call(
        _pool_head_body,
        grid=(GRID,),
        in_specs=[blk] * NCHUNK + [
            pl.BlockSpec((1, 1, ROW_BLK), lambda i: (i, 0, 0)),
            pl.BlockSpec((HID, OUT_C), lambda i: (0, 0)),
            pl.BlockSpec((1, OUT_C), lambda i: (0, 0)),
        ],
        out_specs=pl.BlockSpec((NUM_GRAPHS, OUT_C), lambda i: (0, 0)),
        out_shape=jax.ShapeDtypeStruct((NUM_GRAPHS, OUT_C), jnp.float32),
        scratch_shapes=[
            pltpu.VMEM((NUM_GRAPHS, HID), jnp.float32),
            pltpu.VMEM((NUM_GRAPHS, 1), jnp.float32),
        ],
    )(*hs, batch.reshape(GRID, 1, ROW_BLK), w, b.reshape(1, OUT_C))


def kernel(x, edge_index, batch, lin0_w, lin0_b, conv_ws, lin1_w, lin1_b):
    src = edge_index[0]
    dst = edge_index[1]
    npad = E_PAD - E
    srcb = jnp.concatenate(
        [src, jnp.full((npad,), N, jnp.int32)]).reshape(NTILE, NBATCH, BATCH)
    dstb = jnp.concatenate(
        [dst, jnp.zeros((npad,), jnp.int32)]).reshape(NTILE, NBATCH, BATCH)
    zeros = jnp.zeros((AGG_ROWS, FC), jnp.float32)

    x_pad = jnp.zeros((N_PAD, IN_C), jnp.float32).at[:N].set(x)
    batch_pad = jnp.concatenate(
        [batch, jnp.full((N_PAD - N,), -1, jnp.int32)])

    hs = _lin0(x_pad, lin0_w, lin0_b)
    x0s = hs
    for layer in range(NUM_LAYERS):
        beta = float(math.log(THETA / (layer + 1) + 1.0))
        aggs = _spmm_call(*hs, srcb, dstb, zeros)
        hs = _layer(aggs, x0s, hs, conv_ws[layer], beta)
    return _pool_head(hs, batch_pad, lin1_w, lin1_b)


# speedup vs baseline: 1.2980x; 1.0107x over previous
import functools
import math

import jax
import jax.numpy as jnp
from jax import lax
from jax.experimental import pallas as pl
from jax.experimental.pallas import tpu as pltpu
from jax.experimental.pallas import tpu_sc as plsc

N = 10000
E = 160000
IN_C = 256
HID = 512
OUT_C = 64
NUM_LAYERS = 8
ALPHA = 0.5
THETA = 1.0
NUM_GRAPHS = 128

FC = 128
NCHUNK = HID // FC
NTILE = 16
BATCH = 128
EPT = 10112
NBATCH = EPT // BATCH
E_PAD = EPT * NTILE
N_PAD = 10008
AGG_ROWS = N_PAD
ZMAIN = 624
ZLAST = N_PAD - 15 * ZMAIN

ROW_BLK = 1112
GRID = N_PAD // ROW_BLK


def _spmm_body(*args):
    h_refs = args[0:NCHUNK]
    srcb, dstb, zeros = args[NCHUNK:NCHUNK + 3]
    o_refs = args[NCHUNK + 3:2 * NCHUNK + 3]
    src_v, dst_v, rows, gsem, agg = args[2 * NCHUNK + 3:]
    cid = lax.axis_index("c")
    sid = lax.axis_index("s")
    pltpu.sync_copy(srcb.at[sid], src_v)
    pltpu.sync_copy(dstb.at[sid], dst_v)

    for chunk in range(NCHUNK):
        @pl.when(cid == chunk // (NCHUNK // 2))
        def _(h_ref=h_refs[chunk], o_ref=o_refs[chunk]):
            @pl.when(sid < NTILE - 1)
            def _zero_main():
                pltpu.sync_copy(zeros.at[pl.ds(sid * ZMAIN, ZMAIN)],
                                agg.at[pl.ds(sid * ZMAIN, ZMAIN)])

            @pl.when(sid == NTILE - 1)
            def _zero_last():
                pltpu.sync_copy(zeros.at[pl.ds(15 * ZMAIN, ZLAST)],
                                agg.at[pl.ds(15 * ZMAIN, ZLAST)])

            plsc.subcore_barrier()

            def batch_body(j, carry):
                pltpu.async_copy(h_ref.at[src_v.at[j]], rows, gsem).wait()
                pltpu.sync_copy(rows, agg.at[dst_v.at[j]], add=True)
                return carry

            lax.fori_loop(0, NBATCH, batch_body, 0)
            plsc.subcore_barrier()

            @pl.when(sid < NTILE - 1)
            def _copy_main():
                pltpu.sync_copy(agg.at[pl.ds(sid * ZMAIN, ZMAIN)],
                                o_ref.at[pl.ds(sid * ZMAIN, ZMAIN)])

            @pl.when(sid == NTILE - 1)
            def _copy_last():
                pltpu.sync_copy(agg.at[pl.ds(15 * ZMAIN, ZLAST)],
                                o_ref.at[pl.ds(15 * ZMAIN, ZLAST)])


_spmm_call = pl.kernel(
    _spmm_body,
    out_type=tuple(jax.ShapeDtypeStruct((N_PAD, FC), jnp.float32)
                   for _ in range(NCHUNK)),
    mesh=plsc.VectorSubcoreMesh(core_axis_name="c", subcore_axis_name="s"),
    scratch_types=[
        pltpu.VMEM((NBATCH, BATCH), jnp.int32),
        pltpu.VMEM((NBATCH, BATCH), jnp.int32),
        pltpu.VMEM((BATCH, FC), jnp.float32),
        pltpu.SemaphoreType.DMA,
        pltpu.VMEM_SHARED((AGG_ROWS, FC), jnp.float32),
    ],
)


def _split_store(o_refs, y):
    for j, o in enumerate(o_refs):
        o[...] = y[:, j * FC:(j + 1) * FC]


def _cat(refs):
    return jnp.concatenate([r[...] for r in refs], axis=1)


def _lin0_body(x_ref, w_ref, b_ref, *o_refs):
    y = jnp.maximum(
        jnp.dot(x_ref[...], w_ref[...], preferred_element_type=jnp.float32)
        + b_ref[...], 0.0)
    _split_store(o_refs, y)


def _lin0(x, w, b):
    return pl.pallas_call(
        _lin0_body,
        grid=(GRID,),
        in_specs=[
            pl.BlockSpec((ROW_BLK, IN_C), lambda i: (i, 0)),
            pl.BlockSpec((IN_C, HID), lambda i: (0, 0)),
            pl.BlockSpec((1, HID), lambda i: (0, 0)),
        ],
        out_specs=[pl.BlockSpec((ROW_BLK, FC), lambda i: (i, 0))] * NCHUNK,
        out_shape=[jax.ShapeDtypeStruct((N_PAD, FC), jnp.float32)] * NCHUNK,
    )(x, w, b.reshape(1, HID))


def _layer_body(beta, *refs):
    agg_refs = refs[0:NCHUNK]
    x0_refs = refs[NCHUNK:2 * NCHUNK]
    h_refs = refs[2 * NCHUNK:3 * NCHUNK]
    w_ref = refs[3 * NCHUNK]
    o_refs = refs[3 * NCHUNK + 1:]
    out = _cat(agg_refs) * (1.0 - ALPHA) + ALPHA * _cat(x0_refs)
    y = (1.0 - beta) * out + beta * jnp.dot(
        out, w_ref[...], preferred_element_type=jnp.float32)
    _split_store(o_refs, jnp.maximum(y + _cat(h_refs), 0.0))


def _layer(aggs, x0s, hs, w, beta):
    blk = pl.BlockSpec((ROW_BLK, FC), lambda i: (i, 0))
    return pl.pallas_call(
        functools.partial(_layer_body, beta),
        grid=(GRID,),
        in_specs=[blk] * (3 * NCHUNK)
        + [pl.BlockSpec((HID, HID), lambda i: (0, 0))],
        out_specs=[blk] * NCHUNK,
        out_shape=[jax.ShapeDtypeStruct((N_PAD, FC), jnp.float32)] * NCHUNK,
    )(*aggs, *x0s, *hs, w)


def _pool_head_body(*refs):
    h_refs = refs[0:NCHUNK]
    batch_ref, w_ref, b_ref, o_ref, sums, counts = refs[NCHUNK:]
    i = pl.program_id(0)

    @pl.when(i == 0)
    def _init():
        sums[...] = jnp.zeros_like(sums)
        counts[...] = jnp.zeros_like(counts)

    seg = batch_ref[0]
    gids = lax.broadcasted_iota(jnp.int32, (NUM_GRAPHS, ROW_BLK), 0)
    onehot = (gids == seg).astype(jnp.float32)
    sums[...] += jnp.dot(onehot, _cat(h_refs),
                         preferred_element_type=jnp.float32)
    counts[...] += jnp.sum(onehot, axis=1, keepdims=True)

    @pl.when(i == GRID - 1)
    def _fin():
        pooled = sums[...] / jnp.clip(counts[...], 1.0, None)
        logits = jnp.dot(pooled, w_ref[...],
                         preferred_element_type=jnp.float32) + b_ref[...]
        m = jnp.max(logits, axis=-1, keepdims=True)
        z = logits - m
        lse = jnp.log(jnp.sum(jnp.exp(z), axis=-1, keepdims=True))
        o_ref[...] = z - lse


def _pool_head(hs, batch, w, b):
    blk = pl.BlockSpec((ROW_BLK, FC), lambda i: (i, 0))
    return pl.pallas_call(
        _pool_head_body,
        grid=(GRID,),
        in_specs=[blk] * NCHUNK + [
            pl.BlockSpec((1, 1, ROW_BLK), lambda i: (i, 0, 0)),
            pl.BlockSpec((HID, OUT_C), lambda i: (0, 0)),
            pl.BlockSpec((1, OUT_C), lambda i: (0, 0)),
        ],
        out_specs=pl.BlockSpec((NUM_GRAPHS, OUT_C), lambda i: (0, 0)),
        out_shape=jax.ShapeDtypeStruct((NUM_GRAPHS, OUT_C), jnp.float32),
        scratch_shapes=[
            pltpu.VMEM((NUM_GRAPHS, HID), jnp.float32),
            pltpu.VMEM((NUM_GRAPHS, 1), jnp.float32),
        ],
    )(*hs, batch.reshape(GRID, 1, ROW_BLK), w, b.reshape(1, OUT_C))


def kernel(x, edge_index, batch, lin0_w, lin0_b, conv_ws, lin1_w, lin1_b):
    src = edge_index[0]
    dst = edge_index[1]
    npad = E_PAD - E
    srcb = jnp.concatenate(
        [src, jnp.full((npad,), N, jnp.int32)]).reshape(NTILE, NBATCH, BATCH)
    dstb = jnp.concatenate(
        [dst, jnp.zeros((npad,), jnp.int32)]).reshape(NTILE, NBATCH, BATCH)
    zeros = jnp.zeros((AGG_ROWS, FC), jnp.float32)

    x_pad = jnp.zeros((N_PAD, IN_C), jnp.float32).at[:N].set(x)
    batch_pad = jnp.concatenate(
        [batch, jnp.full((N_PAD - N,), -1, jnp.int32)])

    hs = _lin0(x_pad, lin0_w, lin0_b)
    x0s = hs
    for layer in range(NUM_LAYERS):
        beta = float(math.log(THETA / (layer + 1) + 1.0))
        aggs = _spmm_call(*hs, srcb, dstb, zeros)
        hs = _layer(aggs, x0s, hs, conv_ws[layer], beta)
    return _pool_head(hs, batch_pad, lin1_w, lin1_b)
